# baseline (speedup 1.0000x reference)
import functools

import numpy as np
import jax
import jax.numpy as jnp
from jax import lax
from jax.experimental import pallas as pl
from jax.experimental.pallas import tpu as pltpu
from jax.experimental.pallas import tpu_sc as plsc

B = 4096
D = 64
QL = 20
DL = 200
NC = 2
NS = 16
NW = NC * NS
RPW = B // NW
QIW = RPW * QL
DIW = RPW * DL
CH = 128
QCH = QIW // CH
DCH = DIW // CH


def _seg_pattern(n_chunks: int, seg_len: int) -> np.ndarray:
    pos = np.arange(n_chunks * CH, dtype=np.int32)
    base = (pos // seg_len).reshape(1, n_chunks, CH)
    offs = (np.arange(NS, dtype=np.int32) * RPW).reshape(NS, 1, 1)
    return base + offs


_QPAT = _seg_pattern(QCH, QL)
_DPAT = _seg_pattern(DCH, DL)


def _sc_pool(q_table, d_table, qs_flat, ds_flat, qpat, dpat, zeros):
    mesh = plsc.VectorSubcoreMesh(core_axis_name="c", subcore_axis_name="s")

    @functools.partial(
        pl.kernel,
        out_type=[
            jax.ShapeDtypeStruct((B, D), jnp.float32),
            jax.ShapeDtypeStruct((B, D), jnp.float32),
        ],
        mesh=mesh,
        compiler_params=pltpu.CompilerParams(use_tc_tiling_on_sc=False),
        scratch_types=[
            pltpu.VMEM((QIW,), jnp.int32),
            pltpu.VMEM((DIW,), jnp.int32),
            pltpu.VMEM((QCH, CH), jnp.int32),
            pltpu.VMEM((DCH, CH), jnp.int32),
            pltpu.VMEM((CH, D), jnp.float32),
            pltpu.VMEM_SHARED((NS * RPW, D), jnp.float32),
            pltpu.VMEM_SHARED((NS * RPW, D), jnp.float32),
        ],
    )
    def sc_kernel(qt_hbm, dt_hbm, qi_hbm, di_hbm, qp_hbm, dp_hbm, z_hbm,
                  qsum_hbm, dsum_hbm,
                  qidx_v, didx_v, qpat_v, dpat_v, buf_v, qsh, dsh):
        sid = lax.axis_index("s")
        wid = sid * NC + lax.axis_index("c")

        pltpu.sync_copy(qi_hbm.at[pl.ds(wid * QIW, QIW)], qidx_v)
        pltpu.sync_copy(di_hbm.at[pl.ds(wid * DIW, DIW)], didx_v)
        pltpu.sync_copy(qp_hbm.at[sid], qpat_v)
        pltpu.sync_copy(dp_hbm.at[sid], dpat_v)
        pltpu.sync_copy(z_hbm, qsh.at[pl.ds(sid * RPW, RPW)])
        pltpu.sync_copy(z_hbm, dsh.at[pl.ds(sid * RPW, RPW)])

        @pl.loop(0, QCH)
        def _(c):
            pltpu.sync_copy(qt_hbm.at[qidx_v.at[pl.ds(c * CH, CH)]], buf_v)
            pltpu.sync_copy(buf_v, qsh.at[qpat_v.at[c]], add=True)

        @pl.loop(0, DCH)
        def _(c):
            pltpu.sync_copy(dt_hbm.at[didx_v.at[pl.ds(c * CH, CH)]], buf_v)
            pltpu.sync_copy(buf_v, dsh.at[dpat_v.at[c]], add=True)

        pltpu.sync_copy(qsh.at[pl.ds(sid * RPW, RPW)], qsum_hbm.at[pl.ds(wid * RPW, RPW)])
        pltpu.sync_copy(dsh.at[pl.ds(sid * RPW, RPW)], dsum_hbm.at[pl.ds(wid * RPW, RPW)])

    return sc_kernel(q_table, d_table, qs_flat, ds_flat, qpat, dpat, zeros)


def _tc_finish(q_sum, d_sum):
    def body(qs_ref, ds_ref, o_ref):
        q = jnp.tanh(qs_ref[...] * (1.0 / QL))
        d = jnp.tanh(ds_ref[...] * (1.0 / DL))
        qn = jnp.maximum(jnp.sqrt(jnp.sum(q * q, axis=1, keepdims=True)), 1e-12)
        dn = jnp.maximum(jnp.sqrt(jnp.sum(d * d, axis=1, keepdims=True)), 1e-12)
        o_ref[...] = jnp.sum((q / qn) * (d / dn), axis=1)

    return pl.pallas_call(
        body,
        out_shape=jax.ShapeDtypeStruct((B,), jnp.float32),
    )(q_sum, d_sum)


def kernel(qs, ds, q_table, d_table):
    qs_flat = qs.reshape(-1)
    ds_flat = ds.reshape(-1)
    qpat = jnp.asarray(_QPAT)
    dpat = jnp.asarray(_DPAT)
    zeros = jnp.zeros((RPW, D), jnp.float32)
    q_sum, d_sum = _sc_pool(q_table, d_table, qs_flat, ds_flat, qpat, dpat, zeros)
    return _tc_finish(q_sum, d_sum)
